# baseline (speedup 1.0000x reference)
import functools

import jax
import jax.numpy as jnp
from jax import lax
from jax.experimental import pallas as pl
from jax.experimental.pallas import tpu as pltpu
from jax.experimental.pallas import tpu_sc as plsc

_C = 128
_NC, _NS = 2, 16
_G = 64


def _deg_kernel(npad, per_w):
    m = npad // _NS
    mesh = plsc.VectorSubcoreMesh(core_axis_name="c", subcore_axis_name="s")

    @functools.partial(
        pl.kernel,
        mesh=mesh,
        compiler_params=pltpu.CompilerParams(needs_layout_passes=False),
        out_type=jax.ShapeDtypeStruct((_NC * npad,), jnp.float32),
        scratch_types=[
            pltpu.VMEM((npad,), jnp.float32),
            pltpu.VMEM((per_w, _C), jnp.int32),
            pltpu.VMEM_SHARED((_NS * npad,), jnp.float32),
            pltpu.VMEM((_NS * m,), jnp.float32),
            pltpu.VMEM((m,), jnp.float32),
        ],
    )
    def k(dstp, zrow1d, out, acc, idst, sh, part, red):
        c = lax.axis_index("c")
        s = lax.axis_index("s")
        wid = s * _NC + c
        pltpu.sync_copy(zrow1d, acc)
        pltpu.sync_copy(dstp.at[pl.ds(wid * per_w, per_w)], idst)
        ones = jnp.ones((16,), jnp.float32)

        def body(i, carry):
            for j in range(_C // 16):
                iv = idst[i, pl.ds(j * 16, 16)]
                plsc.addupdate_scatter(acc, [iv], ones)
            return carry

        lax.fori_loop(0, per_w, body, 0)
        pltpu.sync_copy(acc, sh.at[pl.ds(s * npad, npad)])
        plsc.subcore_barrier()
        for p in range(_NS):
            pltpu.sync_copy(sh.at[pl.ds(p * npad + s * m, m)],
                            part.at[pl.ds(p * m, m)])

        def rbody(j, carry):
            v = jnp.zeros((16,), jnp.float32)
            for p in range(_NS):
                v += part[pl.ds(p * m + j * 16, 16)]
            red[pl.ds(j * 16, 16)] = v
            return carry

        lax.fori_loop(0, m // 16, rbody, 0)
        pltpu.sync_copy(red, out.at[pl.ds(c * npad + s * m, m)])

    return k


_GRP = 8


def _msg_kernel(npad, per_w, d):
    rpt = npad // _NS
    mesh = plsc.VectorSubcoreMesh(core_axis_name="c", subcore_axis_name="s")

    @functools.partial(
        pl.kernel,
        mesh=mesh,
        out_type=jax.ShapeDtypeStruct((_NC, npad, d), jnp.float32),
        scratch_types=[
            pltpu.VMEM_SHARED((npad, d), jnp.float32),
            pltpu.VMEM((2, _GRP, _C), jnp.int32),
            pltpu.VMEM((2, _GRP, _C), jnp.int32),
            pltpu.VMEM((2, _C, d), jnp.float32),
            pltpu.SemaphoreType.DMA((2,)),
            pltpu.SemaphoreType.DMA((2,)),
            pltpu.SemaphoreType.DMA((2,)),
        ],
    )
    def k(tab, srcp, dstp, zrow, out, acc, isb, idb, rows, isem, jsem, gsem):
        c = lax.axis_index("c")
        s = lax.axis_index("s")
        wid = s * _NC + c
        pltpu.sync_copy(zrow, acc.at[pl.ds(s * rpt, rpt)])
        plsc.subcore_barrier()

        ngroups = per_w // _GRP

        def idx_start(gbase, q):
            row0 = wid * per_w + gbase
            pltpu.async_copy(srcp.at[pl.ds(row0, _GRP)], isb.at[q],
                             isem.at[q])
            pltpu.async_copy(dstp.at[pl.ds(row0, _GRP)], idb.at[q],
                             jsem.at[q])

        def idx_wait(gbase, q):
            row0 = wid * per_w + gbase
            pltpu.make_async_copy(srcp.at[pl.ds(row0, _GRP)], isb.at[q],
                                  isem.at[q]).wait()
            pltpu.make_async_copy(dstp.at[pl.ds(row0, _GRP)], idb.at[q],
                                  jsem.at[q]).wait()

        def gather_start(b, q, j):
            pltpu.async_copy(tab.at[isb.at[q, j]], rows.at[b], gsem.at[b])

        def gather_wait(b):
            pltpu.make_async_copy(tab.at[isb.at[0, 0]], rows.at[b],
                                  gsem.at[b]).wait()

        idx_start(0, 0)
        idx_start(_GRP, 1)
        idx_wait(0, 0)
        gather_start(0, 0, 0)
        gather_start(1, 0, 1)

        def group_visits(ibase, q, do_start):
            for j in range(_GRP):
                b = j % 2
                gather_wait(b)
                pltpu.sync_copy(rows.at[b], acc.at[idb.at[q, j]], add=True)
                if j < _GRP - 2:
                    gather_start(b, q, j + 2)
                elif j == _GRP - 2:
                    idx_wait(ibase + _GRP, 1 - q)
                    gather_start(b, 1 - q, 0)
                else:
                    gather_start(b, 1 - q, 1)
                    if do_start:
                        idx_start(ibase + 2 * _GRP, q)

        def body(gp, carry):
            ibase = gp * 2 * _GRP
            group_visits(ibase, 0, True)
            group_visits(ibase + _GRP, 1, True)
            return carry

        lax.fori_loop(0, (ngroups - 2) // 2, body, 0)
        group_visits((ngroups - 2) * _GRP, 0, False)
        ibase = (ngroups - 1) * _GRP
        for j in range(_GRP):
            b = j % 2
            gather_wait(b)
            pltpu.sync_copy(rows.at[b], acc.at[idb.at[1, j]], add=True)
            if j < _GRP - 2:
                gather_start(b, 1, j + 2)

        plsc.subcore_barrier()
        pltpu.sync_copy(acc.at[pl.ds(s * rpt, rpt)],
                        out.at[c, pl.ds(s * rpt, rpt)])

    return k


def _tc_matmul(x, W1, bn):
    n, d = x.shape
    h = W1.shape[1]

    def body(x_ref, w_ref, h_ref):
        h_ref[...] = jnp.dot(x_ref[...], w_ref[...],
                             preferred_element_type=jnp.float32)

    return pl.pallas_call(
        body,
        grid=(n // bn,),
        in_specs=[
            pl.BlockSpec((bn, d), lambda i: (i, 0)),
            pl.BlockSpec((d, h), lambda i: (0, 0)),
        ],
        out_specs=pl.BlockSpec((bn, h), lambda i: (i, 0)),
        out_shape=jax.ShapeDtypeStruct((n, h), jnp.float32),
    )(x, W1)


def _col(grid2d, nr):
    return jnp.concatenate(
        [grid2d[r:r + 1, :].reshape(128, 1) for r in range(nr)], axis=0)


def _tc_prep(hm, degp, bn):
    n, h = hm.shape
    nr = bn // 128

    def body(h_ref, degp_ref, g_ref, dis_ref):
        dp = degp_ref[...]
        disg = lax.rsqrt(1.0 + dp[0] + dp[1])
        dis_ref[...] = disg
        g_ref[...] = h_ref[...] * _col(disg, nr)

    return pl.pallas_call(
        body,
        grid=(n // bn,),
        in_specs=[
            pl.BlockSpec((bn, h), lambda i: (i, 0)),
            pl.BlockSpec((_NC, nr, 128), lambda i: (0, i, 0)),
        ],
        out_specs=[
            pl.BlockSpec((bn, h), lambda i: (i, 0)),
            pl.BlockSpec((nr, 128), lambda i: (i, 0)),
        ],
        out_shape=[
            jax.ShapeDtypeStruct((n, h), jnp.float32),
            jax.ShapeDtypeStruct((n // 128, 128), jnp.float32),
        ],
    )(hm, degp)


def _tc_mid(s1, g1, disg, b1, W2, bn):
    n, h = g1.shape
    nr = bn // 128

    def body(s_ref, g_ref, dis_ref, b_ref, w_ref, out_ref):
        sv = s_ref[...]
        col = _col(dis_ref[...], nr)
        t = (sv[0] + sv[1] + g_ref[...]) * col + b_ref[...]
        t = jnp.maximum(t, 0.0)
        h2 = jnp.dot(t, w_ref[...], preferred_element_type=jnp.float32)
        out_ref[...] = h2 * col

    return pl.pallas_call(
        body,
        grid=(n // bn,),
        in_specs=[
            pl.BlockSpec((_NC, bn, h), lambda i: (0, i, 0)),
            pl.BlockSpec((bn, h), lambda i: (i, 0)),
            pl.BlockSpec((nr, 128), lambda i: (i, 0)),
            pl.BlockSpec((1, h), lambda i: (0, 0)),
            pl.BlockSpec((h, h), lambda i: (0, 0)),
        ],
        out_specs=pl.BlockSpec((bn, h), lambda i: (i, 0)),
        out_shape=jax.ShapeDtypeStruct((n, h), jnp.float32),
    )(s1, g1, disg, b1, W2)


def _tc_head(s2, g2, disg, b2, batchg, Wo, bo, bn):
    n, h = g2.shape
    nc = Wo.shape[1]
    nr = bn // 128
    steps = n // bn

    def body(s_ref, g_ref, dis_ref, b_ref, bat_ref, wo_ref, bo_ref, out_ref,
             pool_acc, cnt_acc):
        i = pl.program_id(0)

        @pl.when(i == 0)
        def _():
            pool_acc[...] = jnp.zeros_like(pool_acc)
            cnt_acc[...] = jnp.zeros_like(cnt_acc)

        sv = s_ref[...]
        u = (sv[0] + sv[1] + g_ref[...]) * _col(dis_ref[...], nr) + b_ref[...]
        u = jnp.maximum(u, 0.0)
        bcol = _col(bat_ref[...].astype(jnp.float32), nr)
        gids = lax.broadcasted_iota(jnp.int32, (bn, _G), 1).astype(jnp.float32)
        onehot = (bcol == gids).astype(jnp.float32)
        pool_acc[...] += lax.dot_general(
            onehot, u, (((0,), (0,)), ((), ())),
            preferred_element_type=jnp.float32)
        cnt_acc[...] += jnp.sum(onehot, axis=0)[:, None]

        @pl.when(i == steps - 1)
        def _():
            pooled = pool_acc[...] / jnp.maximum(cnt_acc[...], 1.0)
            logits = jnp.dot(pooled, wo_ref[...],
                             preferred_element_type=jnp.float32) + bo_ref[...]
            m = jnp.max(logits, axis=1, keepdims=True)
            lse = m + jnp.log(jnp.sum(jnp.exp(logits - m), axis=1,
                                      keepdims=True))
            out_ref[...] = logits - lse

    return pl.pallas_call(
        body,
        grid=(steps,),
        in_specs=[
            pl.BlockSpec((_NC, bn, h), lambda i: (0, i, 0)),
            pl.BlockSpec((bn, h), lambda i: (i, 0)),
            pl.BlockSpec((nr, 128), lambda i: (i, 0)),
            pl.BlockSpec((1, h), lambda i: (0, 0)),
            pl.BlockSpec((nr, 128), lambda i: (i, 0)),
            pl.BlockSpec((h, nc), lambda i: (0, 0)),
            pl.BlockSpec((1, nc), lambda i: (0, 0)),
        ],
        out_specs=pl.BlockSpec((_G, nc), lambda i: (0, 0)),
        out_shape=jax.ShapeDtypeStruct((_G, nc), jnp.float32),
        scratch_shapes=[
            pltpu.VMEM((_G, h), jnp.float32),
            pltpu.VMEM((_G, 1), jnp.float32),
        ],
    )(s2, g2, disg, b2, batchg, Wo, bo)


def kernel(x, edge_index, batch, W1, b1, W2, b2, Wo, bo):
    n, d = x.shape
    h = W1.shape[1]
    e = edge_index.shape[1]
    nw = _NC * _NS

    npad = ((n + 1 + 2047) // 2048) * 2048
    nch = -(-e // _C)
    per_w = -(-nch // nw)
    per_w = ((per_w + 2 * _GRP - 1) // (2 * _GRP)) * 2 * _GRP
    epad = nw * per_w * _C
    pad = epad - e

    flat = edge_index.reshape(2 * e)
    src = flat[:e]
    dst = flat[e:]
    if pad:
        ar = jnp.arange(pad, dtype=jnp.int32)
        src = jnp.concatenate([src, ar % n])
        dst = jnp.concatenate([dst, n + ar % (npad - n)])
    src = src.reshape(nw * per_w, _C)
    dst = dst.reshape(nw * per_w, _C)

    zrow = jnp.zeros((npad // _NS, h), jnp.float32)
    zrow1d = jnp.zeros((npad,), jnp.float32)

    bn = 2048
    xp = jnp.concatenate([x, jnp.zeros((npad - n, d), jnp.float32)])
    batchg = jnp.concatenate(
        [batch, jnp.full((npad - n,), _G, jnp.int32)]).reshape(npad // 128,
                                                               128)

    degp = _deg_kernel(npad, per_w)(dst, zrow1d)
    degp = degp.reshape(_NC, npad // 128, 128)
    hm = _tc_matmul(xp, W1, bn)
    g1, disg = _tc_prep(hm, degp, bn)

    msg = _msg_kernel(npad, per_w, h)
    s1 = msg(g1, src, dst, zrow)
    g2 = _tc_mid(s1, g1, disg, b1.reshape(1, -1), W2, bn)
    s2 = msg(g2, src, dst, zrow)
    return _tc_head(s2, g2, disg, b2.reshape(1, -1), batchg,
                    Wo, bo.reshape(1, -1), bn)


# speedup vs baseline: 1.0579x; 1.0579x over previous
import functools

import jax
import jax.numpy as jnp
from jax import lax
from jax.experimental import pallas as pl
from jax.experimental.pallas import tpu as pltpu
from jax.experimental.pallas import tpu_sc as plsc

_C = 128
_NC, _NS = 2, 16
_G = 64


def _deg_kernel(npad, per_w):
    m = npad // _NS
    mesh = plsc.VectorSubcoreMesh(core_axis_name="c", subcore_axis_name="s")

    @functools.partial(
        pl.kernel,
        mesh=mesh,
        compiler_params=pltpu.CompilerParams(needs_layout_passes=False),
        out_type=jax.ShapeDtypeStruct((_NC * npad,), jnp.float32),
        scratch_types=[
            pltpu.VMEM((npad,), jnp.float32),
            pltpu.VMEM((per_w, _C), jnp.int32),
            pltpu.VMEM_SHARED((_NS * npad,), jnp.float32),
            pltpu.VMEM((_NS * m,), jnp.float32),
            pltpu.VMEM((m,), jnp.float32),
        ],
    )
    def k(dstp, zrow1d, out, acc, idst, sh, part, red):
        c = lax.axis_index("c")
        s = lax.axis_index("s")
        wid = s * _NC + c
        pltpu.sync_copy(zrow1d, acc)
        pltpu.sync_copy(dstp.at[pl.ds(wid * per_w, per_w)], idst)
        ones = jnp.ones((16,), jnp.float32)

        def body(i, carry):
            for j in range(_C // 16):
                iv = idst[i, pl.ds(j * 16, 16)]
                plsc.addupdate_scatter(acc, [iv], ones)
            return carry

        lax.fori_loop(0, per_w, body, 0)
        pltpu.sync_copy(acc, sh.at[pl.ds(s * npad, npad)])
        plsc.subcore_barrier()
        for p in range(_NS):
            pltpu.sync_copy(sh.at[pl.ds(p * npad + s * m, m)],
                            part.at[pl.ds(p * m, m)])

        def rbody(j, carry):
            v = jnp.zeros((16,), jnp.float32)
            for p in range(_NS):
                v += part[pl.ds(p * m + j * 16, 16)]
            red[pl.ds(j * 16, 16)] = v
            return carry

        lax.fori_loop(0, m // 16, rbody, 0)
        pltpu.sync_copy(red, out.at[pl.ds(c * npad + s * m, m)])

    return k


_GRP = 8


def _msg_kernel(npad, per_w, d):
    rpt = npad // _NS
    mesh = plsc.VectorSubcoreMesh(core_axis_name="c", subcore_axis_name="s")

    @functools.partial(
        pl.kernel,
        mesh=mesh,
        out_type=jax.ShapeDtypeStruct((_NC, npad, d), jnp.float32),
        scratch_types=[
            pltpu.VMEM_SHARED((npad, d), jnp.float32),
            pltpu.VMEM((2, _GRP * _C), jnp.int32),
            pltpu.VMEM((2, _GRP, _C), jnp.int32),
            pltpu.VMEM((2, _C, d), jnp.float32),
            pltpu.SemaphoreType.DMA((2,)),
            pltpu.SemaphoreType.DMA((2,)),
            pltpu.SemaphoreType.DMA((2,)),
        ],
    )
    def k(tab, srcp, dstp, zrow, out, acc, isb, idb, rows, isem, jsem, gsem):
        c = lax.axis_index("c")
        s = lax.axis_index("s")
        wid = s * _NC + c
        pltpu.sync_copy(zrow, acc.at[pl.ds(s * rpt, rpt)])
        plsc.subcore_barrier()

        ngroups = per_w // _GRP

        def idx_start(gbase, q):
            row0 = wid * per_w + gbase
            pltpu.async_copy(srcp.at[pl.ds(row0 * _C, _GRP * _C)], isb.at[q],
                             isem.at[q])
            pltpu.async_copy(dstp.at[pl.ds(row0, _GRP)], idb.at[q],
                             jsem.at[q])

        def idx_wait(gbase, q):
            row0 = wid * per_w + gbase
            pltpu.make_async_copy(srcp.at[pl.ds(row0 * _C, _GRP * _C)],
                                  isb.at[q], isem.at[q]).wait()
            pltpu.make_async_copy(dstp.at[pl.ds(row0, _GRP)], idb.at[q],
                                  jsem.at[q]).wait()

        def gather_start(b, q, j):
            pltpu.async_copy(tab.at[isb.at[q, pl.ds(j * _C, _C)]], rows.at[b],
                             gsem.at[b])

        def gather_wait(b):
            pltpu.make_async_copy(tab.at[isb.at[0, pl.ds(0, _C)]], rows.at[b],
                                  gsem.at[b]).wait()

        idx_start(0, 0)
        idx_start(_GRP, 1)
        idx_wait(0, 0)
        gather_start(0, 0, 0)
        gather_start(1, 0, 1)

        def group_visits(ibase, q, do_start):
            for j in range(_GRP):
                b = j % 2
                gather_wait(b)
                pltpu.sync_copy(rows.at[b], acc.at[idb.at[q, j]], add=True)
                if j < _GRP - 2:
                    gather_start(b, q, j + 2)
                elif j == _GRP - 2:
                    idx_wait(ibase + _GRP, 1 - q)
                    gather_start(b, 1 - q, 0)
                else:
                    gather_start(b, 1 - q, 1)
                    if do_start:
                        idx_start(ibase + 2 * _GRP, q)

        def body(gp, carry):
            ibase = gp * 2 * _GRP
            group_visits(ibase, 0, True)
            group_visits(ibase + _GRP, 1, True)
            return carry

        lax.fori_loop(0, (ngroups - 2) // 2, body, 0)
        group_visits((ngroups - 2) * _GRP, 0, False)
        ibase = (ngroups - 1) * _GRP
        for j in range(_GRP):
            b = j % 2
            gather_wait(b)
            pltpu.sync_copy(rows.at[b], acc.at[idb.at[1, j]], add=True)
            if j < _GRP - 2:
                gather_start(b, 1, j + 2)

        plsc.subcore_barrier()
        pltpu.sync_copy(acc.at[pl.ds(s * rpt, rpt)],
                        out.at[c, pl.ds(s * rpt, rpt)])

    return k


def _tc_matmul(x, W1, bn):
    n, d = x.shape
    h = W1.shape[1]

    def body(x_ref, w_ref, h_ref):
        h_ref[...] = jnp.dot(x_ref[...], w_ref[...],
                             preferred_element_type=jnp.float32)

    return pl.pallas_call(
        body,
        grid=(n // bn,),
        in_specs=[
            pl.BlockSpec((bn, d), lambda i: (i, 0)),
            pl.BlockSpec((d, h), lambda i: (0, 0)),
        ],
        out_specs=pl.BlockSpec((bn, h), lambda i: (i, 0)),
        out_shape=jax.ShapeDtypeStruct((n, h), jnp.float32),
    )(x, W1)


def _col(grid2d, nr):
    return jnp.concatenate(
        [grid2d[r:r + 1, :].reshape(128, 1) for r in range(nr)], axis=0)


def _tc_prep(hm, degp, bn):
    n, h = hm.shape
    nr = bn // 128

    def body(h_ref, degp_ref, g_ref, dis_ref):
        dp = degp_ref[...]
        disg = lax.rsqrt(1.0 + dp[0] + dp[1])
        dis_ref[...] = disg
        g_ref[...] = h_ref[...] * _col(disg, nr)

    return pl.pallas_call(
        body,
        grid=(n // bn,),
        in_specs=[
            pl.BlockSpec((bn, h), lambda i: (i, 0)),
            pl.BlockSpec((_NC, nr, 128), lambda i: (0, i, 0)),
        ],
        out_specs=[
            pl.BlockSpec((bn, h), lambda i: (i, 0)),
            pl.BlockSpec((nr, 128), lambda i: (i, 0)),
        ],
        out_shape=[
            jax.ShapeDtypeStruct((n, h), jnp.float32),
            jax.ShapeDtypeStruct((n // 128, 128), jnp.float32),
        ],
    )(hm, degp)


def _tc_mid(s1, g1, disg, b1, W2, bn):
    n, h = g1.shape
    nr = bn // 128

    def body(s_ref, g_ref, dis_ref, b_ref, w_ref, out_ref):
        sv = s_ref[...]
        col = _col(dis_ref[...], nr)
        t = (sv[0] + sv[1] + g_ref[...]) * col + b_ref[...]
        t = jnp.maximum(t, 0.0)
        h2 = jnp.dot(t, w_ref[...], preferred_element_type=jnp.float32)
        out_ref[...] = h2 * col

    return pl.pallas_call(
        body,
        grid=(n // bn,),
        in_specs=[
            pl.BlockSpec((_NC, bn, h), lambda i: (0, i, 0)),
            pl.BlockSpec((bn, h), lambda i: (i, 0)),
            pl.BlockSpec((nr, 128), lambda i: (i, 0)),
            pl.BlockSpec((1, h), lambda i: (0, 0)),
            pl.BlockSpec((h, h), lambda i: (0, 0)),
        ],
        out_specs=pl.BlockSpec((bn, h), lambda i: (i, 0)),
        out_shape=jax.ShapeDtypeStruct((n, h), jnp.float32),
    )(s1, g1, disg, b1, W2)


def _tc_head(s2, g2, disg, b2, batchg, Wo, bo, bn):
    n, h = g2.shape
    nc = Wo.shape[1]
    nr = bn // 128
    steps = n // bn

    def body(s_ref, g_ref, dis_ref, b_ref, bat_ref, wo_ref, bo_ref, out_ref,
             pool_acc, cnt_acc):
        i = pl.program_id(0)

        @pl.when(i == 0)
        def _():
            pool_acc[...] = jnp.zeros_like(pool_acc)
            cnt_acc[...] = jnp.zeros_like(cnt_acc)

        sv = s_ref[...]
        u = (sv[0] + sv[1] + g_ref[...]) * _col(dis_ref[...], nr) + b_ref[...]
        u = jnp.maximum(u, 0.0)
        bcol = _col(bat_ref[...].astype(jnp.float32), nr)
        gids = lax.broadcasted_iota(jnp.int32, (bn, _G), 1).astype(jnp.float32)
        onehot = (bcol == gids).astype(jnp.float32)
        pool_acc[...] += lax.dot_general(
            onehot, u, (((0,), (0,)), ((), ())),
            preferred_element_type=jnp.float32)
        cnt_acc[...] += jnp.sum(onehot, axis=0)[:, None]

        @pl.when(i == steps - 1)
        def _():
            pooled = pool_acc[...] / jnp.maximum(cnt_acc[...], 1.0)
            logits = jnp.dot(pooled, wo_ref[...],
                             preferred_element_type=jnp.float32) + bo_ref[...]
            m = jnp.max(logits, axis=1, keepdims=True)
            lse = m + jnp.log(jnp.sum(jnp.exp(logits - m), axis=1,
                                      keepdims=True))
            out_ref[...] = logits - lse

    return pl.pallas_call(
        body,
        grid=(steps,),
        in_specs=[
            pl.BlockSpec((_NC, bn, h), lambda i: (0, i, 0)),
            pl.BlockSpec((bn, h), lambda i: (i, 0)),
            pl.BlockSpec((nr, 128), lambda i: (i, 0)),
            pl.BlockSpec((1, h), lambda i: (0, 0)),
            pl.BlockSpec((nr, 128), lambda i: (i, 0)),
            pl.BlockSpec((h, nc), lambda i: (0, 0)),
            pl.BlockSpec((1, nc), lambda i: (0, 0)),
        ],
        out_specs=pl.BlockSpec((_G, nc), lambda i: (0, 0)),
        out_shape=jax.ShapeDtypeStruct((_G, nc), jnp.float32),
        scratch_shapes=[
            pltpu.VMEM((_G, h), jnp.float32),
            pltpu.VMEM((_G, 1), jnp.float32),
        ],
    )(s2, g2, disg, b2, batchg, Wo, bo)


def kernel(x, edge_index, batch, W1, b1, W2, b2, Wo, bo):
    n, d = x.shape
    h = W1.shape[1]
    e = edge_index.shape[1]
    nw = _NC * _NS

    npad = ((n + 1 + 2047) // 2048) * 2048
    nch = -(-e // _C)
    per_w = -(-nch // nw)
    per_w = ((per_w + 2 * _GRP - 1) // (2 * _GRP)) * 2 * _GRP
    epad = nw * per_w * _C
    pad = epad - e

    flat = edge_index.reshape(2 * e)
    dst = flat[e:]
    if pad:
        ar = jnp.arange(pad, dtype=jnp.int32)
        dst = jnp.concatenate([dst, n + ar % (npad - n)])
    dst = dst.reshape(nw * per_w, _C)

    zrow = jnp.zeros((npad // _NS, h), jnp.float32)
    zrow1d = jnp.zeros((npad,), jnp.float32)

    bn = 2048
    xp = jnp.concatenate([x, jnp.zeros((npad - n, d), jnp.float32)])
    batchg = jnp.concatenate(
        [batch, jnp.full((npad - n,), _G, jnp.int32)]).reshape(npad // 128,
                                                               128)

    degp = _deg_kernel(npad, per_w)(dst, zrow1d)
    degp = degp.reshape(_NC, npad // 128, 128)
    hm = _tc_matmul(xp, W1, bn)
    g1, disg = _tc_prep(hm, degp, bn)

    msg = _msg_kernel(npad, per_w, h)
    s1 = msg(g1, flat, dst, zrow)
    g2 = _tc_mid(s1, g1, disg, b1.reshape(1, -1), W2, bn)
    s2 = msg(g2, flat, dst, zrow)
    return _tc_head(s2, g2, disg, b2.reshape(1, -1), batchg,
                    Wo, bo.reshape(1, -1), bn)
